# baseline (speedup 1.0000x reference)
import jax
import jax.numpy as jnp
from jax.experimental import pallas as pl
from jax.experimental.pallas import tpu as pltpu

_N = 10000
_BM = 200


def _fused_kernel(adj_ref, x_ref, w1_ref, b1_ref, w2_ref, b2_ref,
                  out_ref, s2_ref):
    p = pl.program_id(0)
    i = pl.program_id(1)
    a = adj_ref[...].astype(jnp.bfloat16)

    @pl.when(p == 0)
    def _phase0():
        t = jnp.dot(a, x_ref[...], preferred_element_type=jnp.float32)
        h = jnp.dot(t.astype(jnp.bfloat16), w1_ref[...],
                    preferred_element_type=jnp.float32)
        h = jnp.maximum(h + b1_ref[...], 0.0)
        s2 = jnp.dot(h.astype(jnp.bfloat16), w2_ref[...],
                     preferred_element_type=jnp.float32)
        s2_ref[pl.ds(i * _BM, _BM), :] = s2.astype(jnp.bfloat16)

    @pl.when(p == 1)
    def _phase1():
        o = jnp.dot(a, s2_ref[...], preferred_element_type=jnp.float32)
        out_ref[...] = jnp.maximum(o + b2_ref[...], 0.0)


def kernel(x, adj, W1, b1, W2, b2):
    n, f_in = x.shape
    h1 = W1.shape[1]
    h2 = W2.shape[1]

    x_bf = x.astype(jnp.bfloat16)
    w1_bf = W1.astype(jnp.bfloat16)
    w2_bf = W2.astype(jnp.bfloat16)
    b1_2d = b1.reshape(1, h1)
    b2_2d = b2.reshape(1, h2)

    grid = (2, n // _BM)
    out = pl.pallas_call(
        _fused_kernel,
        grid=grid,
        in_specs=[
            pl.BlockSpec((_BM, _N), lambda p, i: (i, 0)),
            pl.BlockSpec((_N, f_in), lambda p, i: (0, 0)),
            pl.BlockSpec((f_in, h1), lambda p, i: (0, 0)),
            pl.BlockSpec((1, h1), lambda p, i: (0, 0)),
            pl.BlockSpec((h1, h2), lambda p, i: (0, 0)),
            pl.BlockSpec((1, h2), lambda p, i: (0, 0)),
        ],
        out_specs=pl.BlockSpec((_BM, h2), lambda p, i: (i, 0)),
        out_shape=jax.ShapeDtypeStruct((n, h2), jnp.float32),
        scratch_shapes=[pltpu.VMEM((_N, h2), jnp.bfloat16)],
        compiler_params=pltpu.CompilerParams(
            dimension_semantics=("arbitrary", "arbitrary")),
    )(adj, x_bf, w1_bf, b1_2d, w2_bf, b2_2d)

    return out


# speedup vs baseline: 1.1101x; 1.1101x over previous
import jax
import jax.numpy as jnp
from jax.experimental import pallas as pl
from jax.experimental.pallas import tpu as pltpu

_N = 10000
_BM = 200
_NBUF = 4


def _fused_kernel(adj_ref, x_ref, w1_ref, b1_ref, w2_ref, b2_ref,
                  out_ref, abuf, s2_ref, sems):
    nb = _N // _BM
    total = 2 * nb

    def chunk_copy(t):
        row = (t % nb) * _BM
        slot = jax.lax.rem(t, _NBUF)
        return pltpu.make_async_copy(
            adj_ref.at[pl.ds(row, _BM), :],
            abuf.at[slot],
            sems.at[slot],
        )

    for t in range(_NBUF):
        chunk_copy(t).start()

    def load_chunk(t):
        chunk_copy(t).wait()
        return abuf[jax.lax.rem(t, _NBUF)].astype(jnp.bfloat16)

    def phase0_body(t, carry):
        a = load_chunk(t)
        s = jnp.dot(a, x_ref[...], preferred_element_type=jnp.float32)
        h = jnp.dot(s.astype(jnp.bfloat16), w1_ref[...],
                    preferred_element_type=jnp.float32)
        h = jnp.maximum(h + b1_ref[...], 0.0)
        s2 = jnp.dot(h.astype(jnp.bfloat16), w2_ref[...],
                     preferred_element_type=jnp.float32)
        s2_ref[pl.ds(t * _BM, _BM), :] = s2.astype(jnp.bfloat16)
        chunk_copy(t + _NBUF).start()
        return carry

    def phase1_body(t, carry):
        a = load_chunk(t)
        o = jnp.dot(a, s2_ref[...], preferred_element_type=jnp.float32)
        out_ref[pl.ds((t - nb) * _BM, _BM), :] = \
            jnp.maximum(o + b2_ref[...], 0.0)

        @pl.when(t + _NBUF < total)
        def _():
            chunk_copy(t + _NBUF).start()
        return carry

    jax.lax.fori_loop(0, nb, phase0_body, 0, unroll=False)
    jax.lax.fori_loop(nb, total, phase1_body, 0, unroll=False)


def kernel(x, adj, W1, b1, W2, b2):
    n, f_in = x.shape
    h1 = W1.shape[1]
    h2 = W2.shape[1]

    x_bf = x.astype(jnp.bfloat16)
    w1_bf = W1.astype(jnp.bfloat16)
    w2_bf = W2.astype(jnp.bfloat16)
    b1_2d = b1.reshape(1, h1)
    b2_2d = b2.reshape(1, h2)

    vmem = pl.BlockSpec(memory_space=pltpu.MemorySpace.VMEM)
    out = pl.pallas_call(
        _fused_kernel,
        in_specs=[
            pl.BlockSpec(memory_space=pl.ANY),
            vmem, vmem, vmem, vmem, vmem,
        ],
        out_specs=vmem,
        out_shape=jax.ShapeDtypeStruct((n, h2), jnp.float32),
        scratch_shapes=[
            pltpu.VMEM((_NBUF, _BM, _N), jnp.float32),
            pltpu.VMEM((_N, h2), jnp.bfloat16),
            pltpu.SemaphoreType.DMA((_NBUF,)),
        ],
    )(adj, x_bf, w1_bf, b1_2d, w2_bf, b2_2d)

    return out


# speedup vs baseline: 1.1217x; 1.0104x over previous
import jax
import jax.numpy as jnp
from jax.experimental import pallas as pl
from jax.experimental.pallas import tpu as pltpu

_N = 10000
_BM = 200
_NBUF = 5


def _fused_kernel(adj_ref, x_ref, w1_ref, b1_ref, w2_ref, b2_ref,
                  out_ref, abuf, xb_ref, s2f_ref, s2b_ref, sems):
    nb = _N // _BM
    total = 2 * nb

    def chunk_copy(t):
        row = (t % nb) * _BM
        slot = jax.lax.rem(t, _NBUF)
        return pltpu.make_async_copy(
            adj_ref.at[pl.ds(row, _BM), :],
            abuf.at[slot],
            sems.at[slot],
        )

    for t in range(_NBUF):
        chunk_copy(t).start()

    xb_ref[...] = x_ref[...].astype(jnp.bfloat16)

    def load_chunk(t):
        chunk_copy(t).wait()
        return abuf[jax.lax.rem(t, _NBUF)].astype(jnp.bfloat16)

    def phase0_body(t, carry):
        a = load_chunk(t)
        s = jnp.dot(a, xb_ref[...], preferred_element_type=jnp.float32)
        h = jnp.dot(s.astype(jnp.bfloat16), w1_ref[...].astype(jnp.bfloat16),
                    preferred_element_type=jnp.float32)
        h = jnp.maximum(h + b1_ref[...], 0.0)
        s2 = jnp.dot(h.astype(jnp.bfloat16), w2_ref[...].astype(jnp.bfloat16),
                     preferred_element_type=jnp.float32)
        s2f_ref[pl.ds(t * _BM, _BM), :] = s2
        chunk_copy(t + _NBUF).start()
        return carry

    def phase1_body(t, carry):
        a = load_chunk(t)
        o = jnp.dot(a, s2b_ref[...], preferred_element_type=jnp.float32)
        out_ref[pl.ds((t - nb) * _BM, _BM), :] = \
            jnp.maximum(o + b2_ref[...], 0.0)

        @pl.when(t + _NBUF < total)
        def _():
            chunk_copy(t + _NBUF).start()
        return carry

    jax.lax.fori_loop(0, nb, phase0_body, 0, unroll=False)
    s2b_ref[...] = s2f_ref[...].astype(jnp.bfloat16)
    jax.lax.fori_loop(nb, total, phase1_body, 0, unroll=False)


def kernel(x, adj, W1, b1, W2, b2):
    n, f_in = x.shape
    h1 = W1.shape[1]
    h2 = W2.shape[1]

    b1_2d = b1.reshape(1, h1)
    b2_2d = b2.reshape(1, h2)

    vmem = pl.BlockSpec(memory_space=pltpu.MemorySpace.VMEM)
    out = pl.pallas_call(
        _fused_kernel,
        in_specs=[
            pl.BlockSpec(memory_space=pl.ANY),
            vmem, vmem, vmem, vmem, vmem,
        ],
        out_specs=vmem,
        out_shape=jax.ShapeDtypeStruct((n, h2), jnp.float32),
        scratch_shapes=[
            pltpu.VMEM((_NBUF, _BM, _N), jnp.float32),
            pltpu.VMEM((n, f_in), jnp.bfloat16),
            pltpu.VMEM((_N, h2), jnp.float32),
            pltpu.VMEM((_N, h2), jnp.bfloat16),
            pltpu.SemaphoreType.DMA((_NBUF,)),
        ],
    )(adj, x, W1, b1_2d, W2, b2_2d)

    return out
